# baseline (speedup 1.0000x reference)
import functools

import jax
import jax.numpy as jnp
from jax import lax
from jax.experimental import pallas as pl
from jax.experimental.pallas import tpu as pltpu
from jax.experimental.pallas import tpu_sc as plsc

N = 10000
D = 128
E = 320000
NC = 2
NS = 16
NW = NC * NS
EPW = E // NW
CH = 128
EPWP = 10240
PAD = EPWP - EPW
NCHUNK = EPWP // CH
BLKC = 8
NBLK = NCHUNK // BLKC
NA = N + 8
RPS = 624
RTAIL = N - NS * RPS

_BLK = 2000


def _mm_body(x_ref, w_ref, o_ref):
    o_ref[...] = lax.dot_general(
        x_ref[...], w_ref[...], (((1,), (1,)), ((), ())),
        preferred_element_type=jnp.float32)


def _matmul_t(x, w):
    return pl.pallas_call(
        _mm_body,
        grid=(N // _BLK,),
        in_specs=[pl.BlockSpec((_BLK, D), lambda i: (i, 0)),
                  pl.BlockSpec((D, D), lambda i: (0, 0))],
        out_specs=pl.BlockSpec((_BLK, D), lambda i: (i, 0)),
        out_shape=jax.ShapeDtypeStruct((N, D), jnp.float32),
    )(x, w)


def _mid_body(agg_ref, x_ref, wroot_ref, b_ref, wrel2_ref, h_ref, y2_ref):
    h = (agg_ref[0] + agg_ref[1] + b_ref[...] +
         lax.dot_general(x_ref[...], wroot_ref[...], (((1,), (1,)), ((), ())),
                         preferred_element_type=jnp.float32))
    h = jnp.maximum(h, 0.0)
    h_ref[...] = h
    y2_ref[...] = lax.dot_general(
        h, wrel2_ref[...], (((1,), (1,)), ((), ())),
        preferred_element_type=jnp.float32)


def _mid_stage(aggp, x, wroot, b, wrel2):
    return pl.pallas_call(
        _mid_body,
        grid=(N // _BLK,),
        in_specs=[pl.BlockSpec((2, _BLK, D), lambda i: (0, i, 0)),
                  pl.BlockSpec((_BLK, D), lambda i: (i, 0)),
                  pl.BlockSpec((D, D), lambda i: (0, 0)),
                  pl.BlockSpec((1, D), lambda i: (0, 0)),
                  pl.BlockSpec((D, D), lambda i: (0, 0))],
        out_specs=[pl.BlockSpec((_BLK, D), lambda i: (i, 0)),
                   pl.BlockSpec((_BLK, D), lambda i: (i, 0))],
        out_shape=[jax.ShapeDtypeStruct((N, D), jnp.float32),
                   jax.ShapeDtypeStruct((N, D), jnp.float32)],
    )(aggp, x, wroot, b, wrel2)


def _final_body(agg_ref, h_ref, wroot_ref, b_ref, o_ref):
    o_ref[...] = (agg_ref[0] + agg_ref[1] + b_ref[...] +
                  lax.dot_general(h_ref[...], wroot_ref[...],
                                  (((1,), (1,)), ((), ())),
                                  preferred_element_type=jnp.float32))


def _final_stage(aggp, h, wroot, b):
    return pl.pallas_call(
        _final_body,
        grid=(N // _BLK,),
        in_specs=[pl.BlockSpec((2, _BLK, D), lambda i: (0, i, 0)),
                  pl.BlockSpec((_BLK, D), lambda i: (i, 0)),
                  pl.BlockSpec((D, D), lambda i: (0, 0)),
                  pl.BlockSpec((1, D), lambda i: (0, 0))],
        out_specs=pl.BlockSpec((_BLK, D), lambda i: (i, 0)),
        out_shape=jax.ShapeDtypeStruct((N, D), jnp.float32),
    )(aggp, h, wroot, b)


def _sc_segment_sum(y, src, dst, zeros):
    mesh = plsc.VectorSubcoreMesh(core_axis_name="c", subcore_axis_name="s")

    @functools.partial(
        pl.kernel, mesh=mesh,
        out_type=jax.ShapeDtypeStruct((NC, N, D), jnp.float32),
        scratch_types=[
            pltpu.VMEM((NCHUNK, CH), jnp.int32),
            pltpu.VMEM((2 * BLKC, CH), jnp.int32),
            pltpu.VMEM((CH, D), jnp.float32),
            pltpu.VMEM((CH, D), jnp.float32),
            pltpu.VMEM_SHARED((NA, D), jnp.float32),
            pltpu.SemaphoreType.DMA,
            pltpu.SemaphoreType.DMA,
            pltpu.SemaphoreType.DMA,
            pltpu.SemaphoreType.DMA,
        ],
    )
    def scat(y_hbm, src_hbm, dst_hbm, zero_hbm, out_hbm,
             src_v, dstr, rows0, rows1, acc, sem0, sem1, semi0, semi1):
        c = lax.axis_index("c")
        s = lax.axis_index("s")
        wid = s * NC + c
        pltpu.sync_copy(src_hbm.at[wid], src_v)
        pltpu.sync_copy(dst_hbm.at[wid].at[pl.ds(0, 2 * BLKC)], dstr)
        pltpu.sync_copy(zero_hbm.at[pl.ds(s * RPS, RPS)],
                        acc.at[pl.ds(s * RPS, RPS)])

        @pl.when(s == 0)
        def _():
            pltpu.sync_copy(zero_hbm.at[pl.ds(NS * RPS, RTAIL)],
                            acc.at[pl.ds(NS * RPS, RTAIL)])

        plsc.subcore_barrier()

        pltpu.async_copy(y_hbm.at[src_v.at[0]], rows0, sem0)
        pltpu.async_copy(y_hbm.at[src_v.at[1]], rows1, sem1)

        @pl.loop(0, NBLK, step=2)
        def _(b):
            for pb in range(2):
                blk = b + pb
                isem = semi0 if pb == 0 else semi1

                @pl.when(blk >= 2)
                def _():
                    pltpu.make_async_copy(
                        dst_hbm.at[wid].at[pl.ds(blk * BLKC, BLKC)],
                        dstr.at[pl.ds(pb * BLKC, BLKC)], isem).wait()

                for c8 in range(BLKC):
                    j = blk * BLKC + c8
                    row = pb * BLKC + c8
                    rbuf = rows0 if c8 % 2 == 0 else rows1
                    rsem = sem0 if c8 % 2 == 0 else sem1
                    pltpu.make_async_copy(
                        y_hbm.at[src_v.at[j]], rbuf, rsem).wait()
                    pltpu.sync_copy(rbuf, acc.at[dstr.at[row]], add=True)

                    @pl.when(j + 2 < NCHUNK)
                    def _():
                        pltpu.async_copy(
                            y_hbm.at[src_v.at[j + 2]], rbuf, rsem)

                @pl.when(blk + 2 < NBLK)
                def _():
                    pltpu.async_copy(
                        dst_hbm.at[wid].at[pl.ds((blk + 2) * BLKC, BLKC)],
                        dstr.at[pl.ds(pb * BLKC, BLKC)], isem)

        plsc.subcore_barrier()
        pltpu.sync_copy(acc.at[pl.ds(s * RPS, RPS)],
                        out_hbm.at[c].at[pl.ds(s * RPS, RPS)])

        @pl.when(s == 0)
        def _():
            pltpu.sync_copy(acc.at[pl.ds(NS * RPS, RTAIL)],
                            out_hbm.at[c].at[pl.ds(NS * RPS, RTAIL)])

    return scat(y, src, dst, zeros)


def kernel(x, edge_index, W1_rel, b1_rel, W1_root, W2_rel, b2_rel, W2_root):
    ei = edge_index.astype(jnp.int32)
    src = jnp.concatenate(
        [ei[0].reshape(NW, EPW), jnp.zeros((NW, PAD), jnp.int32)],
        axis=1).reshape(NW, NCHUNK, CH)
    dst = jnp.concatenate(
        [ei[1].reshape(NW, EPW), jnp.full((NW, PAD), N, jnp.int32)],
        axis=1).reshape(NW, NCHUNK, CH)
    zeros = jnp.zeros((N, D), jnp.float32)
    b1 = b1_rel.reshape(1, D)
    b2 = b2_rel.reshape(1, D)

    y1 = _matmul_t(x, W1_rel)
    agg1 = _sc_segment_sum(y1, src, dst, zeros)
    h, y2 = _mid_stage(agg1, x, W1_root, b1, W2_rel)
    agg2 = _sc_segment_sum(y2, src, dst, zeros)
    out = _final_stage(agg2, h, W2_root, b2)
    return out
